# baseline (speedup 1.0000x reference)
import jax
import jax.numpy as jnp
from jax import lax
from jax.experimental import pallas as pl
from jax.experimental.pallas import tpu as pltpu
from jax.experimental.pallas import tpu_sc as plsc

F32 = jnp.float32
NC = 2
NS = 16
NW = NC * NS
CH = 80
BE = 512


def _mm(a, b):
    return jnp.dot(a.astype(jnp.bfloat16), b.astype(jnp.bfloat16),
                   preferred_element_type=jnp.float32)


def _softplus(v):
    return jnp.maximum(v, 0.0) + jnp.log(1.0 + jnp.exp(-jnp.abs(v)))


def _emb_select(et, emb_ref, nt):
    res = emb_ref[0:1, :]
    for t in range(1, nt):
        res = jnp.where(et == t, emb_ref[t:t + 1, :], res)
    return res


def kernel(x, pos, eg_w1, eg_b1, eg_w2, eg_b2, eg_emb, el_w1, el_b1, el_w2,
           el_b2, el_emb, sg_lin1, sg_fw1, sg_fb1, sg_fw2, sg_fb2, sg_lin2,
           sg_lb2, gin_w1, gin_b1, gin_w2, gin_b2, gg_w1, gg_b1, gg_w2,
           gg_b2, gl_w1, gl_b1, gl_w2, gl_b2, edge_index, edge_type):
    N, H = x.shape
    E = edge_index.shape[1]
    G = eg_w1.shape[0]
    NT = eg_emb.shape[0]
    EPS = E // NS
    EPW = E // NW
    NCH_S = EPS // CH
    NCH_W = EPW // CH
    NPSA = (N // NS) // 8 * 8
    NPSL = N - (NS - 1) * NPSA
    GE = E // BE
    assert EPS % CH == 0 and EPW % CH == 0 and N % NS == 0 and E % BE == 0
    assert EPW % 16 == 0

    mesh = plsc.VectorSubcoreMesh(core_axis_name="c", subcore_axis_name="s",
                                  num_cores=NC, num_subcores=NS)

    src = edge_index[0].astype(jnp.int32)
    dst = edge_index[1].astype(jnp.int32)
    src16 = src.reshape(NS, NCH_S, CH)
    dst16 = dst.reshape(NS, NCH_S, CH)
    src32 = src.reshape(NW, NCH_W, CH)
    dst32 = dst.reshape(NW, NCH_W, CH)
    srcf = src.reshape(NW, EPW)
    dstf = dst.reshape(NW, EPW)
    et2 = edge_type.reshape(E, 1).astype(jnp.int32)
    pos4 = jnp.pad(pos.astype(F32), ((0, 0), (0, 4 - pos.shape[1])))
    pos4f = pos4.reshape(-1)
    z128 = jnp.zeros((N, H), F32)

    def r2(b):
        return b.reshape(1, -1).astype(F32)

    wga, wgb, wgc = gg_w1[:H], gg_w1[H:2 * H], gg_w1[2 * H:]
    wla, wlb, wlc = gl_w1[:H], gl_w1[H:2 * H], gl_w1[2 * H:]

    ed_f32 = jax.ShapeDtypeStruct((E, H), F32)
    e16_f32 = jax.ShapeDtypeStruct((E, 16), F32)
    e_f32 = jax.ShapeDtypeStruct((E,), F32)
    nd_f32 = jax.ShapeDtypeStruct((N, H), F32)

    eb = pl.BlockSpec((BE, H), lambda i: (i, 0))
    eb16 = pl.BlockSpec((BE, 16), lambda i: (i, 0))
    eb1 = pl.BlockSpec((BE, 1), lambda i: (i, 0))

    def wspec(a):
        zeros = (0,) * a.ndim
        return pl.BlockSpec(a.shape, lambda i, _z=zeros: _z)

    def db_loop(nch, fire, consume):
        fire(0, 0)

        def body(j2, carry):
            j0 = 2 * j2

            @pl.when(j0 + 1 < nch)
            def _():
                fire(j0 + 1, 1)

            consume(j0, 0)

            @pl.when(j0 + 2 < nch)
            def _():
                fire(j0 + 2, 0)

            @pl.when(j0 + 1 < nch)
            def _():
                consume(j0 + 1, 1)

            return carry

        lax.fori_loop(0, (nch + 1) // 2, body, 0)

    def db_gather(tab_h, out_h, idx_v, bufs, sems, nch, ebase):
        def fire(k, which):
            pltpu.async_copy(tab_h.at[idx_v.at[k]], bufs[which], sems[which])

        def consume(k, which):
            pltpu.make_async_copy(tab_h.at[idx_v.at[k]], bufs[which],
                                  sems[which]).wait()
            pltpu.sync_copy(bufs[which], out_h.at[pl.ds(ebase(k), CH)])

        db_loop(nch, fire, consume)

    def scatter_add_loop(m_h, acc, idx_v, buf, nch, ebase):
        def body(j, carry):
            pltpu.sync_copy(m_h.at[pl.ds(ebase(j), CH)], buf)
            pltpu.sync_copy(buf, acc.at[idx_v.at[j]], add=True)
            return carry

        lax.fori_loop(0, nch, body, 0)

    def aligned_writeback(pred, acc, out_ref, s):
        rb = s * NPSA

        @pl.when(pred & (s < NS - 1))
        def _():
            pltpu.sync_copy(acc.at[pl.ds(rb, NPSA)],
                            out_ref.at[pl.ds(rb, NPSA)])

        @pl.when(pred & (s == NS - 1))
        def _():
            pltpu.sync_copy(acc.at[pl.ds(rb, NPSL)],
                            out_ref.at[pl.ds(rb, NPSL)])

    def n0_body(x_r, w_r, hl0_o):
        hl0_o[...] = jnp.dot(x_r[...], w_r[...], preferred_element_type=F32)

    hl0 = pl.pallas_call(n0_body, out_shape=nd_f32)(x, sg_lin1[0])

    def ab_body(dx_r, dy_r, dz_r, et_r, xs_r, hs0_r,
                egw1, egb1, egw2, egb2, egemb,
                elw1, elb1, elw2, elb2, elemb,
                fw1, fb1, fw2, fb2,
                eag_o, eal_o, m0_o, msg_o, ud16_o):
        dxv = dx_r[...]
        dyv = dy_r[...]
        dzv = dz_r[...]
        d = jnp.sqrt(dxv * dxv + dyv * dyv + dzv * dzv + 1e-8)
        step = 10.0 / (G - 1)
        offs = lax.broadcasted_iota(jnp.int32, (1, G), 1).astype(F32) * step
        coeff = -0.5 / step ** 2
        smear = jnp.exp(coeff * (d - offs) ** 2)
        et = et_r[...]

        def enc(w1, b1, w2, b2, emb):
            t = jnp.tanh(_mm(smear, w1[...]) + b1[...])
            t = _mm(t, w2[...]) + b2[...]
            return t * _emb_select(et, emb, NT)

        eag = enc(egw1, egb1, egw2, egb2, egemb)
        eal = enc(elw1, elb1, elw2, elb2, elemb)
        eag_o[...] = eag
        eal_o[...] = eal
        f = _softplus(_mm(eag, fw1[...]) + fb1[...])
        f = _mm(f, fw2[...]) + fb2[...]
        m0_o[...] = hs0_r[...] * f
        msg_o[...] = jnp.maximum(xs_r[...] + eal, 0.0)
        zpad = jnp.zeros((BE, 13), F32)
        ud16_o[...] = jnp.concatenate([dxv, dyv, dzv, zpad], axis=1) / d

    def run_ab(dx2, dy2, dz2, xs, hs0):
        ws = [eg_w1, r2(eg_b1), eg_w2, r2(eg_b2), eg_emb,
              el_w1, r2(el_b1), el_w2, r2(el_b2), el_emb,
              sg_fw1[0], r2(sg_fb1[0]), sg_fw2[0], r2(sg_fb2[0])]
        return pl.pallas_call(
            ab_body,
            grid=(GE,),
            in_specs=[eb1, eb1, eb1, eb1, eb, eb] + [wspec(w) for w in ws],
            out_specs=[eb, eb, eb, eb, eb16],
            out_shape=[ed_f32, ed_f32, ed_f32, ed_f32, e16_f32],
        )(dx2, dy2, dz2, et2, xs, hs0, *ws)

    def c_body(eag_r, hs1_r, fw1, fb1, fw2, fb2, m1_o):
        f = _softplus(_mm(eag_r[...], fw1[...]) + fb1[...])
        f = _mm(f, fw2[...]) + fb2[...]
        m1_o[...] = hs1_r[...] * f

    def run_c(eag, hs1):
        ws = [sg_fw1[1], r2(sg_fb1[1]), sg_fw2[1], r2(sg_fb2[1])]
        return pl.pallas_call(
            c_body,
            grid=(GE,),
            in_specs=[eb, eb] + [wspec(w) for w in ws],
            out_specs=eb,
            out_shape=ed_f32,
        )(eag, hs1, *ws)

    def n1_body(x_r, agg0_r, aggl_r, lin2, lb2, lin1b, gw1, gb1, gw2, gb2,
                wla_r, wlb_r, h1_o, hl1_o, al_o, bl_o):
        h1 = x_r[...] + _softplus(
            jnp.dot(agg0_r[...], lin2[...], preferred_element_type=F32)
            + lb2[...])
        h1_o[...] = h1
        hl1_o[...] = jnp.dot(h1, lin1b[...], preferred_element_type=F32)
        t = jnp.maximum(
            jnp.dot(x_r[...] + aggl_r[...], gw1[...],
                    preferred_element_type=F32) + gb1[...], 0.0)
        hl = jnp.dot(t, gw2[...], preferred_element_type=F32) + gb2[...]
        al_o[...] = jnp.dot(hl, wla_r[...], preferred_element_type=F32)
        bl_o[...] = jnp.dot(hl, wlb_r[...], preferred_element_type=F32)

    def run_n1(agg0, aggl):
        return pl.pallas_call(
            n1_body, out_shape=[nd_f32, nd_f32, nd_f32, nd_f32],
        )(x, agg0, aggl, sg_lin2[0], r2(sg_lb2[0]), sg_lin1[1], gin_w1,
          r2(gin_b1), gin_w2, r2(gin_b2), wla, wlb)

    def n2_body(p_r, h1_r, lin2, lb2, wga_r, wgb_r, ag_o, bg_o):
        agg1 = p_r[0] + p_r[1]
        hg = h1_r[...] + _softplus(
            jnp.dot(agg1, lin2[...], preferred_element_type=F32) + lb2[...])
        ag_o[...] = jnp.dot(hg, wga_r[...], preferred_element_type=F32)
        bg_o[...] = jnp.dot(hg, wgb_r[...], preferred_element_type=F32)

    def run_n2(parts, h1):
        return pl.pallas_call(
            n2_body, out_shape=[nd_f32, nd_f32],
        )(parts, h1, sg_lin2[1], r2(sg_lb2[1]), wga, wgb)

    def d_body(fsg_r, fsl_r, fdg_r, fdl_r, eag_r, eal_r, ud_r,
               wgc_r, ggb1, ggw2r, ggb2, wlc_r, glb1, glw2r, glb2,
               contrib_o):
        sg = jnp.tanh(fsg_r[...] + fdg_r[...]
                      + _mm(eag_r[...], wgc_r[...]) + ggb1[...])
        ig = jnp.sum(sg * ggw2r[...], axis=1, keepdims=True) + ggb2[...]
        sl = jnp.tanh(fsl_r[...] + fdl_r[...]
                      + _mm(eal_r[...], wlc_r[...]) + glb1[...])
        il = jnp.sum(sl * glw2r[...], axis=1, keepdims=True) + glb2[...]
        w = ig + il
        cpad = jnp.zeros((BE, H - 16), F32)
        contrib_o[...] = jnp.concatenate([ud_r[...] * w, cpad], axis=1)

    def run_d(fsg, fsl, fdg, fdl, eag, eal, ud16):
        ws = [wgc, r2(gg_b1), gg_w2.reshape(1, H), gg_b2.reshape(1, 1),
              wlc, r2(gl_b1), gl_w2.reshape(1, H), gl_b2.reshape(1, 1)]
        return pl.pallas_call(
            d_body,
            grid=(GE,),
            in_specs=[eb, eb, eb, eb, eb, eb, eb16]
                     + [wspec(w) for w in ws],
            out_specs=eb,
            out_shape=ed_f32,
        )(fsg, fsl, fdg, fdl, eag, eal, ud16, *ws)

    def g0_body(pos_h, sf_h, df_h, dx_o, dy_o, dz_o,
                pos_v, si_v, di_v, dxb, dyb, dzb):
        c = lax.axis_index("c")
        s = lax.axis_index("s")
        w = s * NC + c
        pltpu.sync_copy(pos_h, pos_v)
        pltpu.sync_copy(sf_h.at[w], si_v)
        pltpu.sync_copy(df_h.at[w], di_v)

        def body(i, carry):
            o = i * 16
            sv = si_v[pl.ds(o, 16)] * 4
            dv = di_v[pl.ds(o, 16)] * 4
            sx = plsc.load_gather(pos_v, [sv])
            sy = plsc.load_gather(pos_v, [sv + 1])
            sz = plsc.load_gather(pos_v, [sv + 2])
            tx = plsc.load_gather(pos_v, [dv])
            ty = plsc.load_gather(pos_v, [dv + 1])
            tz = plsc.load_gather(pos_v, [dv + 2])
            dxb[pl.ds(o, 16)] = tx - sx
            dyb[pl.ds(o, 16)] = ty - sy
            dzb[pl.ds(o, 16)] = tz - sz
            return carry

        lax.fori_loop(0, EPW // 16, body, 0)
        base = w * EPW
        pltpu.sync_copy(dxb, dx_o.at[pl.ds(base, EPW)])
        pltpu.sync_copy(dyb, dy_o.at[pl.ds(base, EPW)])
        pltpu.sync_copy(dzb, dz_o.at[pl.ds(base, EPW)])

    def run_g0():
        return pl.kernel(
            g0_body,
            out_type=[e_f32, e_f32, e_f32],
            mesh=mesh,
            compiler_params=pltpu.CompilerParams(needs_layout_passes=False),
            scratch_types=[
                pltpu.VMEM((4 * N,), F32),
                pltpu.VMEM((EPW,), jnp.int32),
                pltpu.VMEM((EPW,), jnp.int32),
                pltpu.VMEM((EPW,), F32),
                pltpu.VMEM((EPW,), F32),
                pltpu.VMEM((EPW,), F32),
            ],
        )(pos4f, srcf, dstf)

    def g1_body(x_h, hl0_h, s16_h, xs_o, hs0_o,
                idx_v, b0, b1, sm0, sm1):
        c = lax.axis_index("c")
        s = lax.axis_index("s")
        pltpu.sync_copy(s16_h.at[s], idx_v)

        def ebase(k):
            return s * EPS + k * CH

        @pl.when(c == 0)
        def _():
            db_gather(x_h, xs_o, idx_v, (b0, b1), (sm0, sm1), NCH_S, ebase)

        @pl.when(c == 1)
        def _():
            db_gather(hl0_h, hs0_o, idx_v, (b0, b1), (sm0, sm1), NCH_S,
                      ebase)

    def run_g1(hl0_a):
        return pl.kernel(
            g1_body,
            out_type=[ed_f32, ed_f32],
            mesh=mesh,
            scratch_types=[
                pltpu.VMEM((NCH_S, CH), jnp.int32),
                pltpu.VMEM((CH, H), F32),
                pltpu.VMEM((CH, H), F32),
                pltpu.SemaphoreType.DMA,
                pltpu.SemaphoreType.DMA,
            ],
        )(x, hl0_a, src16)

    def s1_body(m0_h, msg_h, d16_h, z_h, agg0_o, aggl_o,
                idx_v, b0, acc):
        c = lax.axis_index("c")
        s = lax.axis_index("s")
        pltpu.sync_copy(d16_h.at[s], idx_v)

        @pl.when(s == 0)
        def _():
            pltpu.sync_copy(z_h, acc)

        plsc.subcore_barrier()

        def ebase(k):
            return s * EPS + k * CH

        @pl.when(c == 0)
        def _():
            scatter_add_loop(m0_h, acc, idx_v, b0, NCH_S, ebase)

        @pl.when(c == 1)
        def _():
            scatter_add_loop(msg_h, acc, idx_v, b0, NCH_S, ebase)

        plsc.subcore_barrier()
        aligned_writeback(c == 0, acc, agg0_o, s)
        aligned_writeback(c == 1, acc, aggl_o, s)

    def run_s1(m0, msg):
        return pl.kernel(
            s1_body,
            out_type=[nd_f32, nd_f32],
            mesh=mesh,
            scratch_types=[
                pltpu.VMEM((NCH_S, CH), jnp.int32),
                pltpu.VMEM((CH, H), F32),
                pltpu.VMEM_SHARED((N, H), F32),
            ],
        )(m0, msg, dst16, z128)

    def g2_body(hl1_h, s32_h, hs1_o, idx_v, b0, b1, sm0, sm1):
        c = lax.axis_index("c")
        s = lax.axis_index("s")
        w = s * NC + c
        pltpu.sync_copy(s32_h.at[w], idx_v)

        def ebase(k):
            return w * EPW + k * CH

        db_gather(hl1_h, hs1_o, idx_v, (b0, b1), (sm0, sm1), NCH_W, ebase)

    def run_g2(hl1):
        return pl.kernel(
            g2_body,
            out_type=ed_f32,
            mesh=mesh,
            scratch_types=[
                pltpu.VMEM((NCH_W, CH), jnp.int32),
                pltpu.VMEM((CH, H), F32),
                pltpu.VMEM((CH, H), F32),
                pltpu.SemaphoreType.DMA,
                pltpu.SemaphoreType.DMA,
            ],
        )(hl1, src32)

    def s2_body(m1_h, d32_h, z_h, part_o, idx_v, b0, acc):
        c = lax.axis_index("c")
        s = lax.axis_index("s")
        w = s * NC + c
        pltpu.sync_copy(d32_h.at[w], idx_v)

        @pl.when(s == 0)
        def _():
            pltpu.sync_copy(z_h, acc)

        plsc.subcore_barrier()

        def ebase(k):
            return w * EPW + k * CH

        scatter_add_loop(m1_h, acc, idx_v, b0, NCH_W, ebase)
        plsc.subcore_barrier()
        rb = s * NPSA

        @pl.when(s < NS - 1)
        def _():
            pltpu.sync_copy(acc.at[pl.ds(rb, NPSA)],
                            part_o.at[c, pl.ds(rb, NPSA)])

        @pl.when(s == NS - 1)
        def _():
            pltpu.sync_copy(acc.at[pl.ds(rb, NPSL)],
                            part_o.at[c, pl.ds(rb, NPSL)])

    def run_s2(m1):
        return pl.kernel(
            s2_body,
            out_type=jax.ShapeDtypeStruct((NC, N, H), F32),
            mesh=mesh,
            scratch_types=[
                pltpu.VMEM((NCH_W, CH), jnp.int32),
                pltpu.VMEM((CH, H), F32),
                pltpu.VMEM_SHARED((N, H), F32),
            ],
        )(m1, dst32, z128)

    def g3_body(ag_h, al_h, bg_h, bl_h, s16_h, d16_h,
                fsg_o, fsl_o, fdg_o, fdl_o,
                idx_a, idx_b, b0, b1, sm0, sm1):
        c = lax.axis_index("c")
        s = lax.axis_index("s")
        pltpu.sync_copy(s16_h.at[s], idx_a)
        pltpu.sync_copy(d16_h.at[s], idx_b)

        def ebase(k):
            return s * EPS + k * CH

        @pl.when(c == 0)
        def _():
            db_gather(ag_h, fsg_o, idx_a, (b0, b1), (sm0, sm1), NCH_S, ebase)
            db_gather(al_h, fsl_o, idx_a, (b0, b1), (sm0, sm1), NCH_S, ebase)

        @pl.when(c == 1)
        def _():
            db_gather(bg_h, fdg_o, idx_b, (b0, b1), (sm0, sm1), NCH_S, ebase)
            db_gather(bl_h, fdl_o, idx_b, (b0, b1), (sm0, sm1), NCH_S, ebase)

    def run_g3(ag, al, bg, bl):
        return pl.kernel(
            g3_body,
            out_type=[ed_f32, ed_f32, ed_f32, ed_f32],
            mesh=mesh,
            scratch_types=[
                pltpu.VMEM((NCH_S, CH), jnp.int32),
                pltpu.VMEM((NCH_S, CH), jnp.int32),
                pltpu.VMEM((CH, H), F32),
                pltpu.VMEM((CH, H), F32),
                pltpu.SemaphoreType.DMA,
                pltpu.SemaphoreType.DMA,
            ],
        )(ag, al, bg, bl, src16, dst16)

    def s3_body(con_h, d16_h, z_h, out_o, idx_v, b0, acc):
        c = lax.axis_index("c")
        s = lax.axis_index("s")

        @pl.when(c == 0)
        def _():
            pltpu.sync_copy(d16_h.at[s], idx_v)

            @pl.when(s == 0)
            def _():
                pltpu.sync_copy(z_h, acc)

        plsc.subcore_barrier()

        def ebase(k):
            return s * EPS + k * CH

        @pl.when(c == 0)
        def _():
            scatter_add_loop(con_h, acc, idx_v, b0, NCH_S, ebase)

        plsc.subcore_barrier()
        aligned_writeback(c == 0, acc, out_o, s)

    def run_s3(contrib):
        return pl.kernel(
            s3_body,
            out_type=nd_f32,
            mesh=mesh,
            scratch_types=[
                pltpu.VMEM((NCH_S, CH), jnp.int32),
                pltpu.VMEM((CH, H), F32),
                pltpu.VMEM_SHARED((N, H), F32),
            ],
        )(contrib, dst16, z128)

    dx, dy, dz = run_g0()
    dx2, dy2, dz2 = dx.reshape(E, 1), dy.reshape(E, 1), dz.reshape(E, 1)
    xs, hs0 = run_g1(hl0)
    eag, eal, m0, msg, ud16 = run_ab(dx2, dy2, dz2, xs, hs0)
    agg0, aggl = run_s1(m0, msg)
    h1, hl1, al, bl = run_n1(agg0, aggl)
    hs1 = run_g2(hl1)
    m1 = run_c(eag, hs1)
    parts = run_s2(m1)
    ag, bg = run_n2(parts, h1)
    fsg, fsl, fdg, fdl = run_g3(ag, al, bg, bl)
    contrib = run_d(fsg, fsl, fdg, fdl, eag, eal, ud16)
    outw = run_s3(contrib)
    return outw[:, :3]


# speedup vs baseline: 1.1585x; 1.1585x over previous
import jax
import jax.numpy as jnp
from jax import lax
from jax.experimental import pallas as pl
from jax.experimental.pallas import tpu as pltpu
from jax.experimental.pallas import tpu_sc as plsc

F32 = jnp.float32
NC = 2
NS = 16
NW = NC * NS
CH = 80
BE = 512


def _mm(a, b):
    return jnp.dot(a.astype(jnp.bfloat16), b.astype(jnp.bfloat16),
                   preferred_element_type=jnp.float32)


def _softplus(v):
    return jnp.maximum(v, 0.0) + jnp.log(1.0 + jnp.exp(-jnp.abs(v)))


def _emb_select(et, emb_ref, nt):
    res = emb_ref[0:1, :]
    for t in range(1, nt):
        res = jnp.where(et == t, emb_ref[t:t + 1, :], res)
    return res


def kernel(x, pos, eg_w1, eg_b1, eg_w2, eg_b2, eg_emb, el_w1, el_b1, el_w2,
           el_b2, el_emb, sg_lin1, sg_fw1, sg_fb1, sg_fw2, sg_fb2, sg_lin2,
           sg_lb2, gin_w1, gin_b1, gin_w2, gin_b2, gg_w1, gg_b1, gg_w2,
           gg_b2, gl_w1, gl_b1, gl_w2, gl_b2, edge_index, edge_type):
    N, H = x.shape
    E = edge_index.shape[1]
    G = eg_w1.shape[0]
    NT = eg_emb.shape[0]
    EPS = E // NS
    EPW = E // NW
    NCH_S = EPS // CH
    NCH_W = EPW // CH
    NPSA = (N // NS) // 8 * 8
    NPSL = N - (NS - 1) * NPSA
    GE = E // BE
    assert EPS % CH == 0 and EPW % CH == 0 and N % NS == 0 and E % BE == 0
    assert EPW % 16 == 0

    mesh = plsc.VectorSubcoreMesh(core_axis_name="c", subcore_axis_name="s",
                                  num_cores=NC, num_subcores=NS)

    src = edge_index[0].astype(jnp.int32)
    dst = edge_index[1].astype(jnp.int32)
    src16 = src.reshape(NS, NCH_S, CH)
    dst16 = dst.reshape(NS, NCH_S, CH)
    src32 = src.reshape(NW, NCH_W, CH)
    dst32 = dst.reshape(NW, NCH_W, CH)
    srcf = src.reshape(NW, EPW)
    dstf = dst.reshape(NW, EPW)
    et2 = edge_type.reshape(E, 1).astype(jnp.int32)
    pos4 = jnp.pad(pos.astype(F32), ((0, 0), (0, 4 - pos.shape[1])))
    pos4f = pos4.reshape(-1)
    z128 = jnp.zeros((N, H), F32)

    def r2(b):
        return b.reshape(1, -1).astype(F32)

    wga, wgb, wgc = gg_w1[:H], gg_w1[H:2 * H], gg_w1[2 * H:]
    wla, wlb, wlc = gl_w1[:H], gl_w1[H:2 * H], gl_w1[2 * H:]

    ed_f32 = jax.ShapeDtypeStruct((E, H), F32)
    e16_f32 = jax.ShapeDtypeStruct((E, 16), F32)
    e_f32 = jax.ShapeDtypeStruct((E,), F32)
    nd_f32 = jax.ShapeDtypeStruct((N, H), F32)

    eb = pl.BlockSpec((BE, H), lambda i: (i, 0))
    eb16 = pl.BlockSpec((BE, 16), lambda i: (i, 0))
    eb1 = pl.BlockSpec((BE, 1), lambda i: (i, 0))

    def wspec(a):
        zeros = (0,) * a.ndim
        return pl.BlockSpec(a.shape, lambda i, _z=zeros: _z)

    def ring_loop(nch, nbuf, fire, consume):
        for b in range(min(nbuf, nch)):
            fire(b, b)

        def body(g, carry):
            j0 = g * nbuf
            for b in range(nbuf):
                @pl.when(j0 + b < nch)
                def _(b=b):
                    consume(j0 + b, b)

                @pl.when(j0 + b + nbuf < nch)
                def _(b=b):
                    fire(j0 + b + nbuf, b)
            return carry

        lax.fori_loop(0, (nch + nbuf - 1) // nbuf, body, 0)

    NBG = 4

    def db_gather(tab_h, out_h, idx_v, bufs, sems, nch, ebase):
        def fire(k, b):
            pltpu.async_copy(tab_h.at[idx_v.at[k]], bufs[b], sems[b])

        def consume(k, b):
            pltpu.make_async_copy(tab_h.at[idx_v.at[k]], bufs[b],
                                  sems[b]).wait()
            pltpu.sync_copy(bufs[b], out_h.at[pl.ds(ebase(k), CH)])

        ring_loop(nch, len(bufs), fire, consume)

    def scatter_add_loop(m_h, dflat_h, acc, dbufs, ibufs, dsems, isems,
                         nch, ebase):
        def fire(k, b):
            pltpu.async_copy(dflat_h.at[pl.ds(ebase(k), CH)], ibufs[b],
                             isems[b])
            pltpu.async_copy(m_h.at[pl.ds(ebase(k), CH)], dbufs[b], dsems[b])

        def consume(k, b):
            pltpu.make_async_copy(dflat_h.at[pl.ds(ebase(k), CH)], ibufs[b],
                                  isems[b]).wait()
            pltpu.make_async_copy(m_h.at[pl.ds(ebase(k), CH)], dbufs[b],
                                  dsems[b]).wait()
            pltpu.sync_copy(dbufs[b], acc.at[ibufs[b]], add=True)

        ring_loop(nch, len(dbufs), fire, consume)

    def aligned_writeback(pred, acc, out_ref, s):
        rb = s * NPSA

        @pl.when(pred & (s < NS - 1))
        def _():
            pltpu.sync_copy(acc.at[pl.ds(rb, NPSA)],
                            out_ref.at[pl.ds(rb, NPSA)])

        @pl.when(pred & (s == NS - 1))
        def _():
            pltpu.sync_copy(acc.at[pl.ds(rb, NPSL)],
                            out_ref.at[pl.ds(rb, NPSL)])

    def n0_body(x_r, w_r, hl0_o):
        hl0_o[...] = jnp.dot(x_r[...], w_r[...], preferred_element_type=F32)

    hl0 = pl.pallas_call(n0_body, out_shape=nd_f32)(x, sg_lin1[0])

    def ab_body(dx_r, dy_r, dz_r, et_r, xs_r, hs0_r,
                egw1, egb1, egw2, egb2, egemb,
                elw1, elb1, elw2, elb2, elemb,
                fw1, fb1, fw2, fb2,
                eag_o, eal_o, m0_o, msg_o, ud16_o):
        dxv = dx_r[...]
        dyv = dy_r[...]
        dzv = dz_r[...]
        d = jnp.sqrt(dxv * dxv + dyv * dyv + dzv * dzv + 1e-8)
        step = 10.0 / (G - 1)
        offs = lax.broadcasted_iota(jnp.int32, (1, G), 1).astype(F32) * step
        coeff = -0.5 / step ** 2
        smear = jnp.exp(coeff * (d - offs) ** 2)
        et = et_r[...]

        def enc(w1, b1, w2, b2, emb):
            t = jnp.tanh(_mm(smear, w1[...]) + b1[...])
            t = _mm(t, w2[...]) + b2[...]
            return t * _emb_select(et, emb, NT)

        eag = enc(egw1, egb1, egw2, egb2, egemb)
        eal = enc(elw1, elb1, elw2, elb2, elemb)
        eag_o[...] = eag
        eal_o[...] = eal
        f = _softplus(_mm(eag, fw1[...]) + fb1[...])
        f = _mm(f, fw2[...]) + fb2[...]
        m0_o[...] = hs0_r[...] * f
        msg_o[...] = jnp.maximum(xs_r[...] + eal, 0.0)
        zpad = jnp.zeros((BE, 13), F32)
        ud16_o[...] = jnp.concatenate([dxv, dyv, dzv, zpad], axis=1) / d

    def run_ab(dx2, dy2, dz2, xs, hs0):
        ws = [eg_w1, r2(eg_b1), eg_w2, r2(eg_b2), eg_emb,
              el_w1, r2(el_b1), el_w2, r2(el_b2), el_emb,
              sg_fw1[0], r2(sg_fb1[0]), sg_fw2[0], r2(sg_fb2[0])]
        return pl.pallas_call(
            ab_body,
            grid=(GE,),
            in_specs=[eb1, eb1, eb1, eb1, eb, eb] + [wspec(w) for w in ws],
            out_specs=[eb, eb, eb, eb, eb16],
            out_shape=[ed_f32, ed_f32, ed_f32, ed_f32, e16_f32],
        )(dx2, dy2, dz2, et2, xs, hs0, *ws)

    def c_body(eag_r, hs1_r, fw1, fb1, fw2, fb2, m1_o):
        f = _softplus(_mm(eag_r[...], fw1[...]) + fb1[...])
        f = _mm(f, fw2[...]) + fb2[...]
        m1_o[...] = hs1_r[...] * f

    def run_c(eag, hs1):
        ws = [sg_fw1[1], r2(sg_fb1[1]), sg_fw2[1], r2(sg_fb2[1])]
        return pl.pallas_call(
            c_body,
            grid=(GE,),
            in_specs=[eb, eb] + [wspec(w) for w in ws],
            out_specs=eb,
            out_shape=ed_f32,
        )(eag, hs1, *ws)

    def n1_body(x_r, agg0_r, aggl_r, lin2, lb2, lin1b, gw1, gb1, gw2, gb2,
                wla_r, wlb_r, h1_o, hl1_o, al_o, bl_o):
        h1 = x_r[...] + _softplus(
            jnp.dot(agg0_r[...], lin2[...], preferred_element_type=F32)
            + lb2[...])
        h1_o[...] = h1
        hl1_o[...] = jnp.dot(h1, lin1b[...], preferred_element_type=F32)
        t = jnp.maximum(
            jnp.dot(x_r[...] + aggl_r[...], gw1[...],
                    preferred_element_type=F32) + gb1[...], 0.0)
        hl = jnp.dot(t, gw2[...], preferred_element_type=F32) + gb2[...]
        al_o[...] = jnp.dot(hl, wla_r[...], preferred_element_type=F32)
        bl_o[...] = jnp.dot(hl, wlb_r[...], preferred_element_type=F32)

    def run_n1(agg0, aggl):
        return pl.pallas_call(
            n1_body, out_shape=[nd_f32, nd_f32, nd_f32, nd_f32],
        )(x, agg0, aggl, sg_lin2[0], r2(sg_lb2[0]), sg_lin1[1], gin_w1,
          r2(gin_b1), gin_w2, r2(gin_b2), wla, wlb)

    def n2_body(p_r, h1_r, lin2, lb2, wga_r, wgb_r, ag_o, bg_o):
        agg1 = p_r[0] + p_r[1]
        hg = h1_r[...] + _softplus(
            jnp.dot(agg1, lin2[...], preferred_element_type=F32) + lb2[...])
        ag_o[...] = jnp.dot(hg, wga_r[...], preferred_element_type=F32)
        bg_o[...] = jnp.dot(hg, wgb_r[...], preferred_element_type=F32)

    def run_n2(parts, h1):
        return pl.pallas_call(
            n2_body, out_shape=[nd_f32, nd_f32],
        )(parts, h1, sg_lin2[1], r2(sg_lb2[1]), wga, wgb)

    def padd_body(p_r, o16):
        o16[...] = p_r[0, :, :16] + p_r[1, :, :16]

    def run_padd(p):
        return pl.pallas_call(
            padd_body, out_shape=jax.ShapeDtypeStruct((N, 16), F32),
        )(p)

    def d_body(fsg_r, fsl_r, fdg_r, fdl_r, eag_r, eal_r, ud_r,
               wgc_r, ggb1, ggw2r, ggb2, wlc_r, glb1, glw2r, glb2,
               contrib_o):
        sg = jnp.tanh(fsg_r[...] + fdg_r[...]
                      + _mm(eag_r[...], wgc_r[...]) + ggb1[...])
        ig = jnp.sum(sg * ggw2r[...], axis=1, keepdims=True) + ggb2[...]
        sl = jnp.tanh(fsl_r[...] + fdl_r[...]
                      + _mm(eal_r[...], wlc_r[...]) + glb1[...])
        il = jnp.sum(sl * glw2r[...], axis=1, keepdims=True) + glb2[...]
        w = ig + il
        cpad = jnp.zeros((BE, H - 16), F32)
        contrib_o[...] = jnp.concatenate([ud_r[...] * w, cpad], axis=1)

    def run_d(fsg, fsl, fdg, fdl, eag, eal, ud16):
        ws = [wgc, r2(gg_b1), gg_w2.reshape(1, H), gg_b2.reshape(1, 1),
              wlc, r2(gl_b1), gl_w2.reshape(1, H), gl_b2.reshape(1, 1)]
        return pl.pallas_call(
            d_body,
            grid=(GE,),
            in_specs=[eb, eb, eb, eb, eb, eb, eb16]
                     + [wspec(w) for w in ws],
            out_specs=eb,
            out_shape=ed_f32,
        )(fsg, fsl, fdg, fdl, eag, eal, ud16, *ws)

    def g0_body(pos_h, sf_h, df_h, dx_o, dy_o, dz_o,
                pos_v, si_v, di_v, dxb, dyb, dzb):
        c = lax.axis_index("c")
        s = lax.axis_index("s")
        w = s * NC + c
        pltpu.sync_copy(pos_h, pos_v)
        pltpu.sync_copy(sf_h.at[w], si_v)
        pltpu.sync_copy(df_h.at[w], di_v)

        def body(i, carry):
            o = i * 16
            sv = si_v[pl.ds(o, 16)] * 4
            dv = di_v[pl.ds(o, 16)] * 4
            sx = plsc.load_gather(pos_v, [sv])
            sy = plsc.load_gather(pos_v, [sv + 1])
            sz = plsc.load_gather(pos_v, [sv + 2])
            tx = plsc.load_gather(pos_v, [dv])
            ty = plsc.load_gather(pos_v, [dv + 1])
            tz = plsc.load_gather(pos_v, [dv + 2])
            dxb[pl.ds(o, 16)] = tx - sx
            dyb[pl.ds(o, 16)] = ty - sy
            dzb[pl.ds(o, 16)] = tz - sz
            return carry

        lax.fori_loop(0, EPW // 16, body, 0)
        base = w * EPW
        pltpu.sync_copy(dxb, dx_o.at[pl.ds(base, EPW)])
        pltpu.sync_copy(dyb, dy_o.at[pl.ds(base, EPW)])
        pltpu.sync_copy(dzb, dz_o.at[pl.ds(base, EPW)])

    def run_g0():
        return pl.kernel(
            g0_body,
            out_type=[e_f32, e_f32, e_f32],
            mesh=mesh,
            compiler_params=pltpu.CompilerParams(needs_layout_passes=False),
            scratch_types=[
                pltpu.VMEM((4 * N,), F32),
                pltpu.VMEM((EPW,), jnp.int32),
                pltpu.VMEM((EPW,), jnp.int32),
                pltpu.VMEM((EPW,), F32),
                pltpu.VMEM((EPW,), F32),
                pltpu.VMEM((EPW,), F32),
            ],
        )(pos4f, srcf, dstf)

    def g1_body(x_h, hl0_h, s16_h, xs_o, hs0_o,
                idx_v, b0, b1, b2, b3, sm0, sm1, sm2, sm3):
        bufs = (b0, b1, b2, b3)
        sems = (sm0, sm1, sm2, sm3)
        c = lax.axis_index("c")
        s = lax.axis_index("s")
        pltpu.sync_copy(s16_h.at[s], idx_v)

        def ebase(k):
            return s * EPS + k * CH

        @pl.when(c == 0)
        def _():
            db_gather(x_h, xs_o, idx_v, bufs, sems, NCH_S, ebase)

        @pl.when(c == 1)
        def _():
            db_gather(hl0_h, hs0_o, idx_v, bufs, sems, NCH_S, ebase)

    def run_g1(hl0_a):
        return pl.kernel(
            g1_body,
            out_type=[ed_f32, ed_f32],
            mesh=mesh,
            scratch_types=[
                pltpu.VMEM((NCH_S, CH), jnp.int32)]
                + [pltpu.VMEM((CH, H), F32)] * NBG
                + [pltpu.SemaphoreType.DMA] * NBG,
        )(x, hl0_a, src16)

    def s1_body(m0_h, msg_h, dflat_h, z_h, agg0_o, aggl_o,
                d0, d1, i0, i1, acc, ds0, ds1, is0, is1):
        c = lax.axis_index("c")
        s = lax.axis_index("s")

        @pl.when(s == 0)
        def _():
            pltpu.sync_copy(z_h, acc)

        plsc.subcore_barrier()

        def ebase(k):
            return s * EPS + k * CH

        @pl.when(c == 0)
        def _():
            scatter_add_loop(m0_h, dflat_h, acc, (d0, d1), (i0, i1),
                             (ds0, ds1), (is0, is1), NCH_S, ebase)

        @pl.when(c == 1)
        def _():
            scatter_add_loop(msg_h, dflat_h, acc, (d0, d1), (i0, i1),
                             (ds0, ds1), (is0, is1), NCH_S, ebase)

        plsc.subcore_barrier()
        aligned_writeback(c == 0, acc, agg0_o, s)
        aligned_writeback(c == 1, acc, aggl_o, s)

    def run_s1(m0, msg):
        return pl.kernel(
            s1_body,
            out_type=[nd_f32, nd_f32],
            mesh=mesh,
            scratch_types=[
                pltpu.VMEM((CH, H), F32),
                pltpu.VMEM((CH, H), F32),
                pltpu.VMEM((CH,), jnp.int32),
                pltpu.VMEM((CH,), jnp.int32),
                pltpu.VMEM_SHARED((N, H), F32),
                pltpu.SemaphoreType.DMA,
                pltpu.SemaphoreType.DMA,
                pltpu.SemaphoreType.DMA,
                pltpu.SemaphoreType.DMA,
            ],
        )(m0, msg, dst, z128)

    def g2_body(hl1_h, s32_h, hs1_o, idx_v, b0, b1, b2, b3,
                sm0, sm1, sm2, sm3):
        c = lax.axis_index("c")
        s = lax.axis_index("s")
        w = s * NC + c
        pltpu.sync_copy(s32_h.at[w], idx_v)

        def ebase(k):
            return w * EPW + k * CH

        db_gather(hl1_h, hs1_o, idx_v, (b0, b1, b2, b3),
                  (sm0, sm1, sm2, sm3), NCH_W, ebase)

    def run_g2(hl1):
        return pl.kernel(
            g2_body,
            out_type=ed_f32,
            mesh=mesh,
            scratch_types=[
                pltpu.VMEM((NCH_W, CH), jnp.int32)]
                + [pltpu.VMEM((CH, H), F32)] * NBG
                + [pltpu.SemaphoreType.DMA] * NBG,
        )(hl1, src32)

    def s2_body(m1_h, dflat_h, z_h, part_o,
                d0, d1, i0, i1, acc, ds0, ds1, is0, is1):
        c = lax.axis_index("c")
        s = lax.axis_index("s")
        w = s * NC + c

        @pl.when(s == 0)
        def _():
            pltpu.sync_copy(z_h, acc)

        plsc.subcore_barrier()

        def ebase(k):
            return w * EPW + k * CH

        scatter_add_loop(m1_h, dflat_h, acc, (d0, d1), (i0, i1),
                         (ds0, ds1), (is0, is1), NCH_W, ebase)
        plsc.subcore_barrier()
        rb = s * NPSA

        @pl.when(s < NS - 1)
        def _():
            pltpu.sync_copy(acc.at[pl.ds(rb, NPSA)],
                            part_o.at[c, pl.ds(rb, NPSA)])

        @pl.when(s == NS - 1)
        def _():
            pltpu.sync_copy(acc.at[pl.ds(rb, NPSL)],
                            part_o.at[c, pl.ds(rb, NPSL)])

    def run_s2(m1):
        return pl.kernel(
            s2_body,
            out_type=jax.ShapeDtypeStruct((NC, N, H), F32),
            mesh=mesh,
            scratch_types=[
                pltpu.VMEM((CH, H), F32),
                pltpu.VMEM((CH, H), F32),
                pltpu.VMEM((CH,), jnp.int32),
                pltpu.VMEM((CH,), jnp.int32),
                pltpu.VMEM_SHARED((N, H), F32),
                pltpu.SemaphoreType.DMA,
                pltpu.SemaphoreType.DMA,
                pltpu.SemaphoreType.DMA,
                pltpu.SemaphoreType.DMA,
            ],
        )(m1, dst, z128)

    def g3_body(ag_h, al_h, bg_h, bl_h, s16_h, d16_h,
                fsg_o, fsl_o, fdg_o, fdl_o,
                idx_a, idx_b, b0, b1, b2, b3, sm0, sm1, sm2, sm3):
        bufs = (b0, b1, b2, b3)
        sems = (sm0, sm1, sm2, sm3)
        c = lax.axis_index("c")
        s = lax.axis_index("s")
        pltpu.sync_copy(s16_h.at[s], idx_a)
        pltpu.sync_copy(d16_h.at[s], idx_b)

        def ebase(k):
            return s * EPS + k * CH

        @pl.when(c == 0)
        def _():
            db_gather(ag_h, fsg_o, idx_a, bufs, sems, NCH_S, ebase)
            db_gather(al_h, fsl_o, idx_a, bufs, sems, NCH_S, ebase)

        @pl.when(c == 1)
        def _():
            db_gather(bg_h, fdg_o, idx_b, bufs, sems, NCH_S, ebase)
            db_gather(bl_h, fdl_o, idx_b, bufs, sems, NCH_S, ebase)

    def run_g3(ag, al, bg, bl):
        return pl.kernel(
            g3_body,
            out_type=[ed_f32, ed_f32, ed_f32, ed_f32],
            mesh=mesh,
            scratch_types=[
                pltpu.VMEM((NCH_S, CH), jnp.int32),
                pltpu.VMEM((NCH_S, CH), jnp.int32)]
                + [pltpu.VMEM((CH, H), F32)] * NBG
                + [pltpu.SemaphoreType.DMA] * NBG,
        )(ag, al, bg, bl, src16, dst16)

    def s3_body(con_h, dflat_h, z_h, part_o,
                d0, d1, i0, i1, acc, ds0, ds1, is0, is1):
        c = lax.axis_index("c")
        s = lax.axis_index("s")
        w = s * NC + c

        @pl.when(s == 0)
        def _():
            pltpu.sync_copy(z_h, acc)

        plsc.subcore_barrier()

        def ebase(k):
            return w * EPW + k * CH

        scatter_add_loop(con_h, dflat_h, acc, (d0, d1), (i0, i1),
                         (ds0, ds1), (is0, is1), NCH_W, ebase)
        plsc.subcore_barrier()
        rb = s * NPSA

        @pl.when(s < NS - 1)
        def _():
            pltpu.sync_copy(acc.at[pl.ds(rb, NPSA)],
                            part_o.at[c, pl.ds(rb, NPSA)])

        @pl.when(s == NS - 1)
        def _():
            pltpu.sync_copy(acc.at[pl.ds(rb, NPSL)],
                            part_o.at[c, pl.ds(rb, NPSL)])

    def run_s3(contrib):
        return pl.kernel(
            s3_body,
            out_type=jax.ShapeDtypeStruct((NC, N, H), F32),
            mesh=mesh,
            scratch_types=[
                pltpu.VMEM((CH, H), F32),
                pltpu.VMEM((CH, H), F32),
                pltpu.VMEM((CH,), jnp.int32),
                pltpu.VMEM((CH,), jnp.int32),
                pltpu.VMEM_SHARED((N, H), F32),
                pltpu.SemaphoreType.DMA,
                pltpu.SemaphoreType.DMA,
                pltpu.SemaphoreType.DMA,
                pltpu.SemaphoreType.DMA,
            ],
        )(contrib, dst, z128)

    dx, dy, dz = run_g0()
    dx2, dy2, dz2 = dx.reshape(E, 1), dy.reshape(E, 1), dz.reshape(E, 1)
    xs, hs0 = run_g1(hl0)
    eag, eal, m0, msg, ud16 = run_ab(dx2, dy2, dz2, xs, hs0)
    agg0, aggl = run_s1(m0, msg)
    h1, hl1, al, bl = run_n1(agg0, aggl)
    hs1 = run_g2(hl1)
    m1 = run_c(eag, hs1)
    parts = run_s2(m1)
    ag, bg = run_n2(parts, h1)
    fsg, fsl, fdg, fdl = run_g3(ag, al, bg, bl)
    contrib = run_d(fsg, fsl, fdg, fdl, eag, eal, ud16)
    outp = run_s3(contrib)
    out16 = run_padd(outp)
    return out16[:, :3]
